# baseline (speedup 1.0000x reference)
import functools

import jax
import jax.numpy as jnp
from jax import lax
from jax.experimental import pallas as pl
from jax.experimental.pallas import tpu as pltpu
from jax.experimental.pallas import tpu_sc as plsc

_NC = 2
_NS = 16
_NW = _NC * _NS

_BN_EPS = 1e-5


def _fps_body(npoint, x_ref, o_ref):
    x = x_ref[...]
    xx, yy, zz = x[:, 0], x[:, 1], x[:, 2]
    f, ns, nl = xx.shape
    flat = (lax.broadcasted_iota(jnp.int32, (f, ns, nl), 1) * nl
            + lax.broadcasted_iota(jnp.int32, (f, ns, nl), 2))
    big = jnp.int32(2 ** 30)

    tw = min(nl, 128)
    lane = lax.broadcasted_iota(jnp.int32, (ns, tw), 1)
    srow = lax.broadcasted_iota(jnp.int32, (ns, tw), 0)
    col = lax.broadcasted_iota(jnp.int32, (1, 1, npoint), 2)

    def _extract(fr, ch, r, c):
        if tw == nl:
            tile = x_ref[fr, ch, :, :]
            c_lo = c
        else:
            c_hi = pl.multiple_of((c // tw) * tw, tw)
            tile = x_ref[fr, ch, :, pl.ds(c_hi, tw)]
            c_lo = c % tw
        return jnp.sum(jnp.where((srow == r) & (lane == c_lo), tile, 0.0))

    def step(i, carry):
        fars, dist, acc = carry
        cs = []
        for fr in range(f):
            r = fars[fr] // nl
            c = fars[fr] % nl
            cs.append(jnp.stack([_extract(fr, 0, r, c),
                                 _extract(fr, 1, r, c),
                                 _extract(fr, 2, r, c)]))
        cbs = jnp.stack(cs).reshape(f, 3, 1)
        cx = cbs[:, 0:1].reshape(f, 1, 1)
        cy = cbs[:, 1:2].reshape(f, 1, 1)
        cz = cbs[:, 2:3].reshape(f, 1, 1)
        d = ((xx - cx) ** 2 + (yy - cy) ** 2) + (zz - cz) ** 2
        dist = jnp.minimum(dist, d)
        acc = jnp.where(col == i, cbs, acc)
        fars2 = []
        for fr in range(f):
            m = jnp.max(dist[fr])
            fars2.append(jnp.min(jnp.where(dist[fr] == m, flat[fr], big)))
        return tuple(fars2), dist, acc

    init = (tuple(jnp.int32(0) for _ in range(f)),
            jnp.full((f, ns, nl), 1e10, jnp.float32),
            jnp.zeros((f, 3, npoint), jnp.float32))
    _, _, acc = lax.fori_loop(0, npoint, step, init)
    o_ref[...] = acc


def _fps(xc, npoint):
    f, _, ns, nl = xc.shape
    return pl.pallas_call(
        functools.partial(_fps_body, npoint),
        out_shape=jax.ShapeDtypeStruct((f, 3, npoint), jnp.float32),
    )(xc)


def _stats_update(acc, s_ref, h, pid):
    @pl.when(pid == 0)
    def _():
        acc[...] = jnp.zeros_like(acc)

    acc[0:1, :] += jnp.sum(h, axis=0, keepdims=True)
    acc[1:2, :] += jnp.sum(h * h, axis=0, keepdims=True)
    s_ref[...] = acc[...]


def _affine(s, g, bt, rcount):
    mean = s[0:1, :] / rcount
    var = s[1:2, :] / rcount - mean * mean
    a = g * lax.rsqrt(var + _BN_EPS)
    return a, bt - mean * a


def _mmp_body(x_ref, w_ref, b_ref, h_ref, s_ref, acc):
    h = lax.dot_general(x_ref[...], w_ref[...], (((0,), (0,)), ((), ())),
                        preferred_element_type=jnp.float32) + b_ref[...]
    h_ref[...] = h
    _stats_update(acc, s_ref, h, pl.program_id(0))


def _mmp(xp, w, b, rb):
    c, r = xp.shape
    d = w.shape[1]
    return pl.pallas_call(
        _mmp_body,
        grid=(r // rb,),
        in_specs=[
            pl.BlockSpec((c, rb), lambda i: (0, i)),
            pl.BlockSpec((c, d), lambda i: (0, 0)),
            pl.BlockSpec((1, d), lambda i: (0, 0)),
        ],
        out_specs=[
            pl.BlockSpec((rb, d), lambda i: (i, 0)),
            pl.BlockSpec((2, d), lambda i: (0, 0)),
        ],
        out_shape=[jax.ShapeDtypeStruct((r, d), jnp.float32),
                   jax.ShapeDtypeStruct((2, d), jnp.float32)],
        scratch_shapes=[pltpu.VMEM((2, d), jnp.float32)],
    )(xp, w, b.reshape(1, d))


def _mmd_body(xp_ref, xr_ref, wa_ref, wb_ref, b_ref, h_ref, s_ref, acc):
    h = (lax.dot_general(xp_ref[...], wa_ref[...], (((0,), (0,)), ((), ())),
                         preferred_element_type=jnp.float32)
         + jnp.dot(xr_ref[...], wb_ref[...], preferred_element_type=jnp.float32)
         + b_ref[...])
    h_ref[...] = h
    _stats_update(acc, s_ref, h, pl.program_id(0))


def _mmd(xp, xr, w, b, rb):
    cp, r = xp.shape
    cr = xr.shape[1]
    d = w.shape[1]
    return pl.pallas_call(
        _mmd_body,
        grid=(r // rb,),
        in_specs=[
            pl.BlockSpec((cp, rb), lambda i: (0, i)),
            pl.BlockSpec((rb, cr), lambda i: (i, 0)),
            pl.BlockSpec((cp, d), lambda i: (0, 0)),
            pl.BlockSpec((cr, d), lambda i: (0, 0)),
            pl.BlockSpec((1, d), lambda i: (0, 0)),
        ],
        out_specs=[
            pl.BlockSpec((rb, d), lambda i: (i, 0)),
            pl.BlockSpec((2, d), lambda i: (0, 0)),
        ],
        out_shape=[jax.ShapeDtypeStruct((r, d), jnp.float32),
                   jax.ShapeDtypeStruct((2, d), jnp.float32)],
        scratch_shapes=[pltpu.VMEM((2, d), jnp.float32)],
    )(xp, xr, w[:cp], w[cp:], b.reshape(1, d))


def _mmr_body(rcount, x_ref, sin_ref, g_ref, bt_ref, w_ref, b_ref,
              h_ref, s_ref, acc):
    a, c = _affine(sin_ref[...], g_ref[...], bt_ref[...], rcount)
    x = jnp.maximum(x_ref[...] * a + c, 0.0)
    h = jnp.dot(x, w_ref[...], preferred_element_type=jnp.float32) + b_ref[...]
    h_ref[...] = h
    _stats_update(acc, s_ref, h, pl.program_id(0))


def _mmr(x, sin, g, bt, w, b, rb):
    r, cin = x.shape
    d = w.shape[1]
    return pl.pallas_call(
        functools.partial(_mmr_body, float(r)),
        grid=(r // rb,),
        in_specs=[
            pl.BlockSpec((rb, cin), lambda i: (i, 0)),
            pl.BlockSpec((2, cin), lambda i: (0, 0)),
            pl.BlockSpec((1, cin), lambda i: (0, 0)),
            pl.BlockSpec((1, cin), lambda i: (0, 0)),
            pl.BlockSpec((cin, d), lambda i: (0, 0)),
            pl.BlockSpec((1, d), lambda i: (0, 0)),
        ],
        out_specs=[
            pl.BlockSpec((rb, d), lambda i: (i, 0)),
            pl.BlockSpec((2, d), lambda i: (0, 0)),
        ],
        out_shape=[jax.ShapeDtypeStruct((r, d), jnp.float32),
                   jax.ShapeDtypeStruct((2, d), jnp.float32)],
        scratch_shapes=[pltpu.VMEM((2, d), jnp.float32)],
    )(x, sin, g.reshape(1, cin), bt.reshape(1, cin), w, b.reshape(1, d))


def _pool_body(rcount, k, x_ref, sin_ref, g_ref, bt_ref, o_ref):
    a, c = _affine(sin_ref[...], g_ref[...], bt_ref[...], rcount)
    h = jnp.maximum(x_ref[...] * a + c, 0.0)
    rb, d = h.shape
    o_ref[...] = jnp.max(h.reshape(rb // k, k, d), axis=1)


def _pool(x, sin, g, bt, k, rb):
    r, d = x.shape
    return pl.pallas_call(
        functools.partial(_pool_body, float(r), k),
        grid=(r // rb,),
        in_specs=[
            pl.BlockSpec((rb, d), lambda i: (i, 0)),
            pl.BlockSpec((2, d), lambda i: (0, 0)),
            pl.BlockSpec((1, d), lambda i: (0, 0)),
            pl.BlockSpec((1, d), lambda i: (0, 0)),
        ],
        out_specs=pl.BlockSpec((rb // k, d), lambda i: (i, 0)),
        out_shape=jax.ShapeDtypeStruct((r // k, d), jnp.float32),
    )(x, sin, g.reshape(1, d), bt.reshape(1, d))


def _bn_relu_full(h, g, bt):
    mean = jnp.mean(h, axis=0, keepdims=True)
    var = jnp.mean(h * h, axis=0, keepdims=True) - mean * mean
    a = g * lax.rsqrt(var + _BN_EPS)
    return jnp.maximum(h * a + (bt - mean * a), 0.0)


def _sa3_body(nf, xp_ref, xr_ref, w1a_ref, w1b_ref, b1_ref, g1_ref, t1_ref,
              w2_ref, b2_ref, g2_ref, t2_ref, w3_ref, b3_ref, g3_ref, t3_ref,
              o_ref):
    h = (lax.dot_general(xp_ref[...], w1a_ref[...], (((0,), (0,)), ((), ())),
                         preferred_element_type=jnp.float32)
         + jnp.dot(xr_ref[...], w1b_ref[...], preferred_element_type=jnp.float32)
         + b1_ref[...])
    h = _bn_relu_full(h, g1_ref[...], t1_ref[...])
    h = jnp.dot(h, w2_ref[...], preferred_element_type=jnp.float32) + b2_ref[...]
    h = _bn_relu_full(h, g2_ref[...], t2_ref[...])
    h = jnp.dot(h, w3_ref[...], preferred_element_type=jnp.float32) + b3_ref[...]
    h = _bn_relu_full(h, g3_ref[...], t3_ref[...])
    r, d = h.shape
    o_ref[...] = jnp.max(h.reshape(nf, r // nf, d), axis=1)


def _sa3(xp, xr, params, nf):
    (w1, b1, g1, t1), (w2, b2, g2, t2), (w3, b3, g3, t3) = params
    r = xr.shape[0]
    d = w3.shape[1]
    args = [xp, xr, w1[:3], w1[3:], b1.reshape(1, -1), g1.reshape(1, -1),
            t1.reshape(1, -1), w2, b2.reshape(1, -1), g2.reshape(1, -1),
            t2.reshape(1, -1), w3, b3.reshape(1, -1), g3.reshape(1, -1),
            t3.reshape(1, -1)]
    return pl.pallas_call(
        functools.partial(_sa3_body, nf),
        out_shape=jax.ShapeDtypeStruct((nf, d), jnp.float32),
    )(*args)


def _mesh():
    return plsc.VectorSubcoreMesh(core_axis_name="c", subcore_axis_name="s")


def _rnd16(v):
    b = plsc.bitcast(v, jnp.uint32)
    b = (b + jnp.uint32(0x7FFF) + ((b >> jnp.uint32(16)) & jnp.uint32(1))) & jnp.uint32(0xFFFF0000)
    return plsc.bitcast(b, jnp.float32)


def _ball1(ptsT, cent):
    f, _, n = ptsT.shape
    s = cent.shape[2]
    k = 32
    r2 = 0.2 ** 2
    wpf = _NW // f
    cpw = s // wpf
    nblk = n // 16

    @functools.partial(
        pl.kernel, mesh=_mesh(),
        compiler_params=pltpu.CompilerParams(needs_layout_passes=False),
        out_type=jax.ShapeDtypeStruct((f * 4 * s * k,), jnp.float32),
        scratch_types=[pltpu.VMEM((n,), jnp.float32)] * 4
                      + [pltpu.VMEM((max(cpw, 128),), jnp.float32)] * 3
                      + [pltpu.VMEM((cpw * k,), jnp.float32)] * 4
                      + [pltpu.VMEM((128,), jnp.int32)],
    )
    def run(pts_h, cent_h, g_h, px, py, pz, pt, cx, cy, cz,
            ob0, ob1, ob2, ob3, ib):
        wid = lax.axis_index("s") * _NC + lax.axis_index("c")
        fr = wid // wpf
        c0 = (wid % wpf) * cpw
        pltpu.sync_copy(pts_h.at[pl.ds((fr * 4 + 0) * n, n)], px)
        pltpu.sync_copy(pts_h.at[pl.ds((fr * 4 + 1) * n, n)], py)
        pltpu.sync_copy(pts_h.at[pl.ds((fr * 4 + 2) * n, n)], pz)
        pltpu.sync_copy(pts_h.at[pl.ds((fr * 4 + 3) * n, n)], pt)
        pltpu.sync_copy(cent_h.at[pl.ds((fr * 3 + 0) * s + c0, cpw)], cx.at[pl.ds(0, cpw)])
        pltpu.sync_copy(cent_h.at[pl.ds((fr * 3 + 1) * s + c0, cpw)], cy.at[pl.ds(0, cpw)])
        pltpu.sync_copy(cent_h.at[pl.ds((fr * 3 + 2) * s + c0, cpw)], cz.at[pl.ds(0, cpw)])
        iota = lax.iota(jnp.int32, 16)
        zeros16 = jnp.zeros((16,), jnp.int32)

        def per_cent(ci, _):
            cis = zeros16 + ci
            cxs = plsc.load_gather(cx, [cis])
            cys = plsc.load_gather(cy, [cis])
            czs = plsc.load_gather(cz, [cis])
            ns = (cxs * cxs + cys * cys) + czs * czs
            cxb = _rnd16(cxs)
            cyb = _rnd16(cys)
            czb = _rnd16(czs)

            def cond(st):
                j, cntv = st
                return jnp.logical_and(jnp.any(cntv < k), j < nblk // 4)

            def bodyw(st):
                j, cntv = st
                off = cntv
                for b in range(4):
                    base = j * 64 + b * 16
                    pidx = base + iota
                    xv = px[pl.ds(base, 16)]
                    yv = py[pl.ds(base, 16)]
                    zv = pz[pl.ds(base, 16)]
                    nd = (xv * xv + yv * yv) + zv * zv
                    dot = (_rnd16(xv) * cxb + _rnd16(yv) * cyb) + _rnd16(zv) * czb
                    d = (ns + nd) - 2.0 * dot
                    mask = d <= r2
                    mi = mask.astype(jnp.int32)
                    inc = plsc.cumsum(mi)
                    plsc.store_scatter(ib, [off + (inc - mi)], pidx, mask=mask)
                    off = off + plsc.all_reduce_population_count(mask)
                return j + 1, off

            _, cnt = lax.while_loop(
                cond, bodyw, (jnp.int32(0), jnp.zeros((16,), jnp.int32)))
            first = plsc.load_gather(ib, [jnp.where(cnt < 0, iota, zeros16)])
            for h in range(k // 16):
                pos = h * 16 + iota
                iv = plsc.load_gather(ib, [pos])
                sel = jnp.where(pos >= cnt, first, iv)
                opos = ci * k + pos
                plsc.store_scatter(ob0, [opos], plsc.load_gather(px, [sel]) - cxs)
                plsc.store_scatter(ob1, [opos], plsc.load_gather(py, [sel]) - cys)
                plsc.store_scatter(ob2, [opos], plsc.load_gather(pz, [sel]) - czs)
                plsc.store_scatter(ob3, [opos], plsc.load_gather(pt, [sel]))
            return 0

        lax.fori_loop(0, cpw, per_cent, 0)
        sk = s * k
        pltpu.sync_copy(ob0, g_h.at[pl.ds((fr * 4 + 0) * sk + c0 * k, cpw * k)])
        pltpu.sync_copy(ob1, g_h.at[pl.ds((fr * 4 + 1) * sk + c0 * k, cpw * k)])
        pltpu.sync_copy(ob2, g_h.at[pl.ds((fr * 4 + 2) * sk + c0 * k, cpw * k)])
        pltpu.sync_copy(ob3, g_h.at[pl.ds((fr * 4 + 3) * sk + c0 * k, cpw * k)])

    return run(ptsT.reshape(-1), cent.reshape(-1)).reshape(f, 4, s * k)


def _ball2(xyzT, cent):
    f, _, n = xyzT.shape
    s = cent.shape[2]
    k = 64
    r2 = 0.4 ** 2
    wpf = _NW // f
    cpw = s // wpf
    nblk = n // 16

    @functools.partial(
        pl.kernel, mesh=_mesh(),
        compiler_params=pltpu.CompilerParams(needs_layout_passes=False),
        out_type=[jax.ShapeDtypeStruct((f * 3 * s * k,), jnp.float32),
                  jax.ShapeDtypeStruct((f * s * k,), jnp.int32)],
        scratch_types=[pltpu.VMEM((n,), jnp.float32)] * 3
                      + [pltpu.VMEM((max(cpw, 128),), jnp.float32)] * 3
                      + [pltpu.VMEM((cpw * k,), jnp.float32)] * 3
                      + [pltpu.VMEM((cpw * k,), jnp.int32),
                         pltpu.VMEM((128,), jnp.int32)],
    )
    def run(pts_h, cent_h, g_h, ix_h, px, py, pz, cx, cy, cz,
            ob0, ob1, ob2, obi, ib):
        wid = lax.axis_index("s") * _NC + lax.axis_index("c")
        fr = wid // wpf
        c0 = (wid % wpf) * cpw
        pltpu.sync_copy(pts_h.at[pl.ds((fr * 3 + 0) * n, n)], px)
        pltpu.sync_copy(pts_h.at[pl.ds((fr * 3 + 1) * n, n)], py)
        pltpu.sync_copy(pts_h.at[pl.ds((fr * 3 + 2) * n, n)], pz)
        pltpu.sync_copy(cent_h.at[pl.ds((fr * 3 + 0) * s + c0, cpw)], cx.at[pl.ds(0, cpw)])
        pltpu.sync_copy(cent_h.at[pl.ds((fr * 3 + 1) * s + c0, cpw)], cy.at[pl.ds(0, cpw)])
        pltpu.sync_copy(cent_h.at[pl.ds((fr * 3 + 2) * s + c0, cpw)], cz.at[pl.ds(0, cpw)])
        iota = lax.iota(jnp.int32, 16)
        zeros16 = jnp.zeros((16,), jnp.int32)

        def per_cent(ci, _):
            cis = zeros16 + ci
            cxs = plsc.load_gather(cx, [cis])
            cys = plsc.load_gather(cy, [cis])
            czs = plsc.load_gather(cz, [cis])
            ns = (cxs * cxs + cys * cys) + czs * czs
            cxb = _rnd16(cxs)
            cyb = _rnd16(cys)
            czb = _rnd16(czs)

            def cond(st):
                j, cntv = st
                return jnp.logical_and(jnp.any(cntv < k), j < nblk // 4)

            def bodyw(st):
                j, cntv = st
                off = cntv
                for b in range(4):
                    base = j * 64 + b * 16
                    pidx = base + iota
                    xv = px[pl.ds(base, 16)]
                    yv = py[pl.ds(base, 16)]
                    zv = pz[pl.ds(base, 16)]
                    nd = (xv * xv + yv * yv) + zv * zv
                    dot = (_rnd16(xv) * cxb + _rnd16(yv) * cyb) + _rnd16(zv) * czb
                    d = (ns + nd) - 2.0 * dot
                    mask = d <= r2
                    mi = mask.astype(jnp.int32)
                    inc = plsc.cumsum(mi)
                    plsc.store_scatter(ib, [off + (inc - mi)], pidx, mask=mask)
                    off = off + plsc.all_reduce_population_count(mask)
                return j + 1, off

            _, cnt = lax.while_loop(
                cond, bodyw, (jnp.int32(0), jnp.zeros((16,), jnp.int32)))
            first = plsc.load_gather(ib, [jnp.where(cnt < 0, iota, zeros16)])
            for h in range(k // 16):
                pos = h * 16 + iota
                iv = plsc.load_gather(ib, [pos])
                sel = jnp.where(pos >= cnt, first, iv)
                opos = ci * k + pos
                plsc.store_scatter(ob0, [opos], plsc.load_gather(px, [sel]) - cxs)
                plsc.store_scatter(ob1, [opos], plsc.load_gather(py, [sel]) - cys)
                plsc.store_scatter(ob2, [opos], plsc.load_gather(pz, [sel]) - czs)
                plsc.store_scatter(obi, [opos], sel + fr * n)
            return 0

        lax.fori_loop(0, cpw, per_cent, 0)
        sk = s * k
        pltpu.sync_copy(ob0, g_h.at[pl.ds((fr * 3 + 0) * sk + c0 * k, cpw * k)])
        pltpu.sync_copy(ob1, g_h.at[pl.ds((fr * 3 + 1) * sk + c0 * k, cpw * k)])
        pltpu.sync_copy(ob2, g_h.at[pl.ds((fr * 3 + 2) * sk + c0 * k, cpw * k)])
        pltpu.sync_copy(obi, ix_h.at[pl.ds(wid * cpw * k, cpw * k)])

    g_out, ix_out = run(xyzT.reshape(-1), cent.reshape(-1))
    return g_out.reshape(f, 3, s * k), ix_out


def _gatherk(table, idx):
    v, d = table.shape
    b = idx.shape[0]
    bpw = b // _NW
    chunk = 128
    nch = bpw // chunk

    @functools.partial(
        pl.kernel, mesh=_mesh(),
        compiler_params=pltpu.CompilerParams(needs_layout_passes=False),
        out_type=jax.ShapeDtypeStruct((b, d), jnp.float32),
        scratch_types=[pltpu.VMEM((chunk,), jnp.int32),
                       pltpu.VMEM((chunk, d), jnp.float32),
                       pltpu.SemaphoreType.DMA],
    )
    def run(tbl_h, idx_h, out_h, idx_v, rows_v, sem):
        wid = lax.axis_index("s") * _NC + lax.axis_index("c")
        base = wid * bpw

        def body(i, _):
            off = base + i * chunk
            pltpu.sync_copy(idx_h.at[pl.ds(off, chunk)], idx_v)
            pltpu.async_copy(tbl_h.at[idx_v], rows_v, sem).wait()
            pltpu.sync_copy(rows_v, out_h.at[pl.ds(off, chunk)])
            return 0

        lax.fori_loop(0, nch, body, 0)

    return run(table, idx)


def kernel(xyz_time, params):
    b, t, n, _ = xyz_time.shape
    f = b * t
    xt = xyz_time.reshape(f, n, 4)
    ptsT = jnp.transpose(xt, (0, 2, 1))
    xyzT = ptsT[:, :3]

    s1, k1 = 512, 32
    nx1 = _fps(xyzT.reshape(f, 3, 8, n // 8), s1)
    g1 = _ball1(ptsT, nx1)
    x1p = jnp.transpose(g1, (1, 0, 2)).reshape(4, f * s1 * k1)
    (w1, b1, g1p, t1p), (w2, b2, g2p, t2p), (w3, b3, g3p, t3p) = params['sa1']
    h1, st1 = _mmp(x1p, w1, b1, rb=2048)
    h2, st2 = _mmr(h1, st1, g1p, t1p, w2, b2, rb=2048)
    h3, st3 = _mmr(h2, st2, g2p, t2p, w3, b3, rb=2048)
    l1 = _pool(h3, st3, g3p, t3p, k=k1, rb=2048)

    s2, k2 = 128, 64
    nx2 = _fps(nx1.reshape(f, 3, 8, s1 // 8), s2)
    g2, idxg = _ball2(nx1, nx2)
    feats = _gatherk(l1, idxg)
    x2p = jnp.transpose(g2, (1, 0, 2)).reshape(3, f * s2 * k2)
    (w1, b1, g1p, t1p), (w2, b2, g2p, t2p), (w3, b3, g3p, t3p) = params['sa2']
    h1, st1 = _mmd(x2p, feats, w1, b1, rb=2048)
    h2, st2 = _mmr(h1, st1, g1p, t1p, w2, b2, rb=2048)
    h3, st3 = _mmr(h2, st2, g2p, t2p, w3, b3, rb=2048)
    l2 = _pool(h3, st3, g3p, t3p, k=k2, rb=2048)

    x3p = jnp.transpose(nx2, (1, 0, 2)).reshape(3, f * s2)
    out = _sa3(x3p, l2, params['sa3'], nf=f)
    return out.reshape(b, t, -1)


# speedup vs baseline: 1.6290x; 1.6290x over previous
import functools

import jax
import jax.numpy as jnp
from jax import lax
from jax.experimental import pallas as pl
from jax.experimental.pallas import tpu as pltpu
from jax.experimental.pallas import tpu_sc as plsc

_NC = 2
_NS = 16
_NW = _NC * _NS

_BN_EPS = 1e-5


def _fps_body(npoint, x_ref, o_ref):
    x = x_ref[...]
    xx, yy, zz = x[:, 0], x[:, 1], x[:, 2]
    f, ns, nl = xx.shape
    flat = (lax.broadcasted_iota(jnp.int32, (f, ns, nl), 1) * nl
            + lax.broadcasted_iota(jnp.int32, (f, ns, nl), 2))
    big = jnp.int32(2 ** 30)

    col = lax.broadcasted_iota(jnp.int32, (1, 1, npoint), 2)
    row = lax.broadcasted_iota(jnp.int32, (1, 3, 1), 1)

    def step(i, carry):
        far, dist, acc = carry
        selm = flat == far
        cx = jnp.sum(jnp.where(selm, xx, 0.0), axis=(1, 2), keepdims=True)
        cy = jnp.sum(jnp.where(selm, yy, 0.0), axis=(1, 2), keepdims=True)
        cz = jnp.sum(jnp.where(selm, zz, 0.0), axis=(1, 2), keepdims=True)
        d = ((xx - cx) ** 2 + (yy - cy) ** 2) + (zz - cz) ** 2
        dist = jnp.minimum(dist, d)
        m = jnp.max(dist, axis=(1, 2), keepdims=True)
        far2 = jnp.min(jnp.where(dist == m, flat, big), axis=(1, 2), keepdims=True)
        cb = jnp.where(row == 0, cx, jnp.where(row == 1, cy, cz))
        acc = jnp.where(col == i, cb, acc)
        return far2, dist, acc

    init = (jnp.zeros((f, 1, 1), jnp.int32),
            jnp.full((f, ns, nl), 1e10, jnp.float32),
            jnp.zeros((f, 3, npoint), jnp.float32))
    _, _, acc = lax.fori_loop(0, npoint, step, init)
    o_ref[...] = acc


def _fps(xc, npoint):
    f, _, ns, nl = xc.shape
    return pl.pallas_call(
        functools.partial(_fps_body, npoint),
        out_shape=jax.ShapeDtypeStruct((f, 3, npoint), jnp.float32),
    )(xc)


def _stats_update(acc, s_ref, h, pid):
    @pl.when(pid == 0)
    def _():
        acc[...] = jnp.zeros_like(acc)

    acc[0:1, :] += jnp.sum(h, axis=0, keepdims=True)
    acc[1:2, :] += jnp.sum(h * h, axis=0, keepdims=True)
    s_ref[...] = acc[...]


def _affine(s, g, bt, rcount):
    mean = s[0:1, :] / rcount
    var = s[1:2, :] / rcount - mean * mean
    a = g * lax.rsqrt(var + _BN_EPS)
    return a, bt - mean * a


def _mmp_body(x_ref, w_ref, b_ref, h_ref, s_ref, acc):
    h = lax.dot_general(x_ref[...], w_ref[...], (((0,), (0,)), ((), ())),
                        preferred_element_type=jnp.float32) + b_ref[...]
    h_ref[...] = h
    _stats_update(acc, s_ref, h, pl.program_id(0))


def _mmp(xp, w, b, rb):
    c, r = xp.shape
    d = w.shape[1]
    return pl.pallas_call(
        _mmp_body,
        grid=(r // rb,),
        in_specs=[
            pl.BlockSpec((c, rb), lambda i: (0, i)),
            pl.BlockSpec((c, d), lambda i: (0, 0)),
            pl.BlockSpec((1, d), lambda i: (0, 0)),
        ],
        out_specs=[
            pl.BlockSpec((rb, d), lambda i: (i, 0)),
            pl.BlockSpec((2, d), lambda i: (0, 0)),
        ],
        out_shape=[jax.ShapeDtypeStruct((r, d), jnp.float32),
                   jax.ShapeDtypeStruct((2, d), jnp.float32)],
        scratch_shapes=[pltpu.VMEM((2, d), jnp.float32)],
    )(xp, w, b.reshape(1, d))


def _mmd_body(xp_ref, xr_ref, wa_ref, wb_ref, b_ref, h_ref, s_ref, acc):
    h = (lax.dot_general(xp_ref[...], wa_ref[...], (((0,), (0,)), ((), ())),
                         preferred_element_type=jnp.float32)
         + jnp.dot(xr_ref[...], wb_ref[...], preferred_element_type=jnp.float32)
         + b_ref[...])
    h_ref[...] = h
    _stats_update(acc, s_ref, h, pl.program_id(0))


def _mmd(xp, xr, w, b, rb):
    cp, r = xp.shape
    cr = xr.shape[1]
    d = w.shape[1]
    return pl.pallas_call(
        _mmd_body,
        grid=(r // rb,),
        in_specs=[
            pl.BlockSpec((cp, rb), lambda i: (0, i)),
            pl.BlockSpec((rb, cr), lambda i: (i, 0)),
            pl.BlockSpec((cp, d), lambda i: (0, 0)),
            pl.BlockSpec((cr, d), lambda i: (0, 0)),
            pl.BlockSpec((1, d), lambda i: (0, 0)),
        ],
        out_specs=[
            pl.BlockSpec((rb, d), lambda i: (i, 0)),
            pl.BlockSpec((2, d), lambda i: (0, 0)),
        ],
        out_shape=[jax.ShapeDtypeStruct((r, d), jnp.float32),
                   jax.ShapeDtypeStruct((2, d), jnp.float32)],
        scratch_shapes=[pltpu.VMEM((2, d), jnp.float32)],
    )(xp, xr, w[:cp], w[cp:], b.reshape(1, d))


def _mmr_body(rcount, x_ref, sin_ref, g_ref, bt_ref, w_ref, b_ref,
              h_ref, s_ref, acc):
    a, c = _affine(sin_ref[...], g_ref[...], bt_ref[...], rcount)
    x = jnp.maximum(x_ref[...] * a + c, 0.0)
    h = jnp.dot(x, w_ref[...], preferred_element_type=jnp.float32) + b_ref[...]
    h_ref[...] = h
    _stats_update(acc, s_ref, h, pl.program_id(0))


def _mmr(x, sin, g, bt, w, b, rb):
    r, cin = x.shape
    d = w.shape[1]
    return pl.pallas_call(
        functools.partial(_mmr_body, float(r)),
        grid=(r // rb,),
        in_specs=[
            pl.BlockSpec((rb, cin), lambda i: (i, 0)),
            pl.BlockSpec((2, cin), lambda i: (0, 0)),
            pl.BlockSpec((1, cin), lambda i: (0, 0)),
            pl.BlockSpec((1, cin), lambda i: (0, 0)),
            pl.BlockSpec((cin, d), lambda i: (0, 0)),
            pl.BlockSpec((1, d), lambda i: (0, 0)),
        ],
        out_specs=[
            pl.BlockSpec((rb, d), lambda i: (i, 0)),
            pl.BlockSpec((2, d), lambda i: (0, 0)),
        ],
        out_shape=[jax.ShapeDtypeStruct((r, d), jnp.float32),
                   jax.ShapeDtypeStruct((2, d), jnp.float32)],
        scratch_shapes=[pltpu.VMEM((2, d), jnp.float32)],
    )(x, sin, g.reshape(1, cin), bt.reshape(1, cin), w, b.reshape(1, d))


def _pool_body(rcount, k, x_ref, sin_ref, g_ref, bt_ref, o_ref):
    a, c = _affine(sin_ref[...], g_ref[...], bt_ref[...], rcount)
    h = jnp.maximum(x_ref[...] * a + c, 0.0)
    rb, d = h.shape
    o_ref[...] = jnp.max(h.reshape(rb // k, k, d), axis=1)


def _pool(x, sin, g, bt, k, rb):
    r, d = x.shape
    return pl.pallas_call(
        functools.partial(_pool_body, float(r), k),
        grid=(r // rb,),
        in_specs=[
            pl.BlockSpec((rb, d), lambda i: (i, 0)),
            pl.BlockSpec((2, d), lambda i: (0, 0)),
            pl.BlockSpec((1, d), lambda i: (0, 0)),
            pl.BlockSpec((1, d), lambda i: (0, 0)),
        ],
        out_specs=pl.BlockSpec((rb // k, d), lambda i: (i, 0)),
        out_shape=jax.ShapeDtypeStruct((r // k, d), jnp.float32),
    )(x, sin, g.reshape(1, d), bt.reshape(1, d))


def _bn_relu_full(h, g, bt):
    mean = jnp.mean(h, axis=0, keepdims=True)
    var = jnp.mean(h * h, axis=0, keepdims=True) - mean * mean
    a = g * lax.rsqrt(var + _BN_EPS)
    return jnp.maximum(h * a + (bt - mean * a), 0.0)


def _sa3_body(nf, xp_ref, xr_ref, w1a_ref, w1b_ref, b1_ref, g1_ref, t1_ref,
              w2_ref, b2_ref, g2_ref, t2_ref, w3_ref, b3_ref, g3_ref, t3_ref,
              o_ref):
    h = (lax.dot_general(xp_ref[...], w1a_ref[...], (((0,), (0,)), ((), ())),
                         preferred_element_type=jnp.float32)
         + jnp.dot(xr_ref[...], w1b_ref[...], preferred_element_type=jnp.float32)
         + b1_ref[...])
    h = _bn_relu_full(h, g1_ref[...], t1_ref[...])
    h = jnp.dot(h, w2_ref[...], preferred_element_type=jnp.float32) + b2_ref[...]
    h = _bn_relu_full(h, g2_ref[...], t2_ref[...])
    h = jnp.dot(h, w3_ref[...], preferred_element_type=jnp.float32) + b3_ref[...]
    h = _bn_relu_full(h, g3_ref[...], t3_ref[...])
    r, d = h.shape
    o_ref[...] = jnp.max(h.reshape(nf, r // nf, d), axis=1)


def _sa3(xp, xr, params, nf):
    (w1, b1, g1, t1), (w2, b2, g2, t2), (w3, b3, g3, t3) = params
    r = xr.shape[0]
    d = w3.shape[1]
    args = [xp, xr, w1[:3], w1[3:], b1.reshape(1, -1), g1.reshape(1, -1),
            t1.reshape(1, -1), w2, b2.reshape(1, -1), g2.reshape(1, -1),
            t2.reshape(1, -1), w3, b3.reshape(1, -1), g3.reshape(1, -1),
            t3.reshape(1, -1)]
    return pl.pallas_call(
        functools.partial(_sa3_body, nf),
        out_shape=jax.ShapeDtypeStruct((nf, d), jnp.float32),
    )(*args)


def _mesh():
    return plsc.VectorSubcoreMesh(core_axis_name="c", subcore_axis_name="s")


def _rnd16(v):
    b = plsc.bitcast(v, jnp.uint32)
    b = (b + jnp.uint32(0x7FFF) + ((b >> jnp.uint32(16)) & jnp.uint32(1))) & jnp.uint32(0xFFFF0000)
    return plsc.bitcast(b, jnp.float32)


def _ball1(ptsT, cent):
    f, _, n = ptsT.shape
    s = cent.shape[2]
    k = 32
    r2 = 0.2 ** 2
    wpf = _NW // f
    cpw = s // wpf
    nblk = n // 16

    @functools.partial(
        pl.kernel, mesh=_mesh(),
        compiler_params=pltpu.CompilerParams(needs_layout_passes=False),
        out_type=jax.ShapeDtypeStruct((f * 4 * s * k,), jnp.float32),
        scratch_types=[pltpu.VMEM((n,), jnp.float32)] * 8
                      + [pltpu.VMEM((max(cpw, 128),), jnp.float32)] * 3
                      + [pltpu.VMEM((cpw * k,), jnp.float32)] * 4
                      + [pltpu.VMEM((128,), jnp.int32)],
    )
    def run(pts_h, cent_h, g_h, px, py, pz, pt, bx, by, bz, nd_s,
            cx, cy, cz, ob0, ob1, ob2, ob3, ib):
        wid = lax.axis_index("s") * _NC + lax.axis_index("c")
        fr = wid // wpf
        c0 = (wid % wpf) * cpw
        pltpu.sync_copy(pts_h.at[pl.ds((fr * 4 + 0) * n, n)], px)
        pltpu.sync_copy(pts_h.at[pl.ds((fr * 4 + 1) * n, n)], py)
        pltpu.sync_copy(pts_h.at[pl.ds((fr * 4 + 2) * n, n)], pz)
        pltpu.sync_copy(pts_h.at[pl.ds((fr * 4 + 3) * n, n)], pt)
        pltpu.sync_copy(cent_h.at[pl.ds((fr * 3 + 0) * s + c0, cpw)], cx.at[pl.ds(0, cpw)])
        pltpu.sync_copy(cent_h.at[pl.ds((fr * 3 + 1) * s + c0, cpw)], cy.at[pl.ds(0, cpw)])
        pltpu.sync_copy(cent_h.at[pl.ds((fr * 3 + 2) * s + c0, cpw)], cz.at[pl.ds(0, cpw)])
        iota = lax.iota(jnp.int32, 16)
        zeros16 = jnp.zeros((16,), jnp.int32)

        def pre(jj, _):
            sl = pl.ds(jj * 16, 16)
            xv, yv, zv = px[sl], py[sl], pz[sl]
            bx[sl] = _rnd16(xv)
            by[sl] = _rnd16(yv)
            bz[sl] = _rnd16(zv)
            nd_s[sl] = (xv * xv + yv * yv) + zv * zv
            return 0

        lax.fori_loop(0, nblk, pre, 0)

        def per_cent(ci, _):
            cis = zeros16 + ci
            cxs = plsc.load_gather(cx, [cis])
            cys = plsc.load_gather(cy, [cis])
            czs = plsc.load_gather(cz, [cis])
            ns = (cxs * cxs + cys * cys) + czs * czs
            cxb = _rnd16(cxs)
            cyb = _rnd16(cys)
            czb = _rnd16(czs)

            def cond(st):
                j, cntv = st
                return jnp.logical_and(jnp.any(cntv < k), j < nblk // 4)

            def bodyw(st):
                j, cntv = st
                off = cntv
                for b in range(4):
                    base = j * 64 + b * 16
                    sl = pl.ds(base, 16)
                    pidx = base + iota
                    dot = (bx[sl] * cxb + by[sl] * cyb) + bz[sl] * czb
                    d = (ns + nd_s[sl]) - 2.0 * dot
                    mask = d <= r2
                    mi = mask.astype(jnp.int32)
                    inc = plsc.cumsum(mi)
                    plsc.store_scatter(ib, [off + (inc - mi)], pidx, mask=mask)
                    off = off + plsc.all_reduce_population_count(mask)
                return j + 1, off

            _, cnt = lax.while_loop(
                cond, bodyw, (jnp.int32(0), jnp.zeros((16,), jnp.int32)))
            first = plsc.load_gather(ib, [jnp.where(cnt < 0, iota, zeros16)])
            for h in range(k // 16):
                pos = h * 16 + iota
                iv = plsc.load_gather(ib, [pos])
                sel = jnp.where(pos >= cnt, first, iv)
                opos = ci * k + pos
                plsc.store_scatter(ob0, [opos], plsc.load_gather(px, [sel]) - cxs)
                plsc.store_scatter(ob1, [opos], plsc.load_gather(py, [sel]) - cys)
                plsc.store_scatter(ob2, [opos], plsc.load_gather(pz, [sel]) - czs)
                plsc.store_scatter(ob3, [opos], plsc.load_gather(pt, [sel]))
            return 0

        lax.fori_loop(0, cpw, per_cent, 0)
        sk = s * k
        pltpu.sync_copy(ob0, g_h.at[pl.ds((fr * 4 + 0) * sk + c0 * k, cpw * k)])
        pltpu.sync_copy(ob1, g_h.at[pl.ds((fr * 4 + 1) * sk + c0 * k, cpw * k)])
        pltpu.sync_copy(ob2, g_h.at[pl.ds((fr * 4 + 2) * sk + c0 * k, cpw * k)])
        pltpu.sync_copy(ob3, g_h.at[pl.ds((fr * 4 + 3) * sk + c0 * k, cpw * k)])

    return run(ptsT.reshape(-1), cent.reshape(-1)).reshape(f, 4, s * k)


def _ball2(xyzT, cent):
    f, _, n = xyzT.shape
    s = cent.shape[2]
    k = 64
    r2 = 0.4 ** 2
    wpf = _NW // f
    cpw = s // wpf
    nblk = n // 16

    @functools.partial(
        pl.kernel, mesh=_mesh(),
        compiler_params=pltpu.CompilerParams(needs_layout_passes=False),
        out_type=[jax.ShapeDtypeStruct((f * 3 * s * k,), jnp.float32),
                  jax.ShapeDtypeStruct((f * s * k,), jnp.int32)],
        scratch_types=[pltpu.VMEM((n,), jnp.float32)] * 7
                      + [pltpu.VMEM((max(cpw, 128),), jnp.float32)] * 3
                      + [pltpu.VMEM((cpw * k,), jnp.float32)] * 3
                      + [pltpu.VMEM((cpw * k,), jnp.int32),
                         pltpu.VMEM((128,), jnp.int32)],
    )
    def run(pts_h, cent_h, g_h, ix_h, px, py, pz, bx, by, bz, nd_s,
            cx, cy, cz, ob0, ob1, ob2, obi, ib):
        wid = lax.axis_index("s") * _NC + lax.axis_index("c")
        fr = wid // wpf
        c0 = (wid % wpf) * cpw
        pltpu.sync_copy(pts_h.at[pl.ds((fr * 3 + 0) * n, n)], px)
        pltpu.sync_copy(pts_h.at[pl.ds((fr * 3 + 1) * n, n)], py)
        pltpu.sync_copy(pts_h.at[pl.ds((fr * 3 + 2) * n, n)], pz)
        pltpu.sync_copy(cent_h.at[pl.ds((fr * 3 + 0) * s + c0, cpw)], cx.at[pl.ds(0, cpw)])
        pltpu.sync_copy(cent_h.at[pl.ds((fr * 3 + 1) * s + c0, cpw)], cy.at[pl.ds(0, cpw)])
        pltpu.sync_copy(cent_h.at[pl.ds((fr * 3 + 2) * s + c0, cpw)], cz.at[pl.ds(0, cpw)])
        iota = lax.iota(jnp.int32, 16)
        zeros16 = jnp.zeros((16,), jnp.int32)

        def pre(jj, _):
            sl = pl.ds(jj * 16, 16)
            xv, yv, zv = px[sl], py[sl], pz[sl]
            bx[sl] = _rnd16(xv)
            by[sl] = _rnd16(yv)
            bz[sl] = _rnd16(zv)
            nd_s[sl] = (xv * xv + yv * yv) + zv * zv
            return 0

        lax.fori_loop(0, nblk, pre, 0)

        def per_cent(ci, _):
            cis = zeros16 + ci
            cxs = plsc.load_gather(cx, [cis])
            cys = plsc.load_gather(cy, [cis])
            czs = plsc.load_gather(cz, [cis])
            ns = (cxs * cxs + cys * cys) + czs * czs
            cxb = _rnd16(cxs)
            cyb = _rnd16(cys)
            czb = _rnd16(czs)

            def cond(st):
                j, cntv = st
                return jnp.logical_and(jnp.any(cntv < k), j < nblk // 4)

            def bodyw(st):
                j, cntv = st
                off = cntv
                for b in range(4):
                    base = j * 64 + b * 16
                    sl = pl.ds(base, 16)
                    pidx = base + iota
                    dot = (bx[sl] * cxb + by[sl] * cyb) + bz[sl] * czb
                    d = (ns + nd_s[sl]) - 2.0 * dot
                    mask = d <= r2
                    mi = mask.astype(jnp.int32)
                    inc = plsc.cumsum(mi)
                    plsc.store_scatter(ib, [off + (inc - mi)], pidx, mask=mask)
                    off = off + plsc.all_reduce_population_count(mask)
                return j + 1, off

            _, cnt = lax.while_loop(
                cond, bodyw, (jnp.int32(0), jnp.zeros((16,), jnp.int32)))
            first = plsc.load_gather(ib, [jnp.where(cnt < 0, iota, zeros16)])
            for h in range(k // 16):
                pos = h * 16 + iota
                iv = plsc.load_gather(ib, [pos])
                sel = jnp.where(pos >= cnt, first, iv)
                opos = ci * k + pos
                plsc.store_scatter(ob0, [opos], plsc.load_gather(px, [sel]) - cxs)
                plsc.store_scatter(ob1, [opos], plsc.load_gather(py, [sel]) - cys)
                plsc.store_scatter(ob2, [opos], plsc.load_gather(pz, [sel]) - czs)
                plsc.store_scatter(obi, [opos], sel + fr * n)
            return 0

        lax.fori_loop(0, cpw, per_cent, 0)
        sk = s * k
        pltpu.sync_copy(ob0, g_h.at[pl.ds((fr * 3 + 0) * sk + c0 * k, cpw * k)])
        pltpu.sync_copy(ob1, g_h.at[pl.ds((fr * 3 + 1) * sk + c0 * k, cpw * k)])
        pltpu.sync_copy(ob2, g_h.at[pl.ds((fr * 3 + 2) * sk + c0 * k, cpw * k)])
        pltpu.sync_copy(obi, ix_h.at[pl.ds(wid * cpw * k, cpw * k)])

    g_out, ix_out = run(xyzT.reshape(-1), cent.reshape(-1))
    return g_out.reshape(f, 3, s * k), ix_out


def _gatherk(table, idx):
    v, d = table.shape
    b = idx.shape[0]
    bpw = b // _NW
    chunk = 128
    nch = bpw // chunk

    @functools.partial(
        pl.kernel, mesh=_mesh(),
        compiler_params=pltpu.CompilerParams(needs_layout_passes=False),
        out_type=jax.ShapeDtypeStruct((b, d), jnp.float32),
        scratch_types=[pltpu.VMEM((chunk,), jnp.int32),
                       pltpu.VMEM((chunk, d), jnp.float32),
                       pltpu.SemaphoreType.DMA],
    )
    def run(tbl_h, idx_h, out_h, idx_v, rows_v, sem):
        wid = lax.axis_index("s") * _NC + lax.axis_index("c")
        base = wid * bpw

        def body(i, _):
            off = base + i * chunk
            pltpu.sync_copy(idx_h.at[pl.ds(off, chunk)], idx_v)
            pltpu.async_copy(tbl_h.at[idx_v], rows_v, sem).wait()
            pltpu.sync_copy(rows_v, out_h.at[pl.ds(off, chunk)])
            return 0

        lax.fori_loop(0, nch, body, 0)

    return run(table, idx)


def kernel(xyz_time, params):
    b, t, n, _ = xyz_time.shape
    f = b * t
    xt = xyz_time.reshape(f, n, 4)
    ptsT = jnp.transpose(xt, (0, 2, 1))
    xyzT = ptsT[:, :3]

    s1, k1 = 512, 32
    nx1 = _fps(xyzT.reshape(f, 3, 8, n // 8), s1)
    g1 = _ball1(ptsT, nx1)
    x1p = jnp.transpose(g1, (1, 0, 2)).reshape(4, f * s1 * k1)
    (w1, b1, g1p, t1p), (w2, b2, g2p, t2p), (w3, b3, g3p, t3p) = params['sa1']
    h1, st1 = _mmp(x1p, w1, b1, rb=2048)
    h2, st2 = _mmr(h1, st1, g1p, t1p, w2, b2, rb=2048)
    h3, st3 = _mmr(h2, st2, g2p, t2p, w3, b3, rb=2048)
    l1 = _pool(h3, st3, g3p, t3p, k=k1, rb=2048)

    s2, k2 = 128, 64
    nx2 = _fps(nx1.reshape(f, 3, 8, s1 // 8), s2)
    g2, idxg = _ball2(nx1, nx2)
    feats = _gatherk(l1, idxg)
    x2p = jnp.transpose(g2, (1, 0, 2)).reshape(3, f * s2 * k2)
    (w1, b1, g1p, t1p), (w2, b2, g2p, t2p), (w3, b3, g3p, t3p) = params['sa2']
    h1, st1 = _mmd(x2p, feats, w1, b1, rb=2048)
    h2, st2 = _mmr(h1, st1, g1p, t1p, w2, b2, rb=2048)
    h3, st3 = _mmr(h2, st2, g2p, t2p, w3, b3, rb=2048)
    l2 = _pool(h3, st3, g3p, t3p, k=k2, rb=2048)

    x3p = jnp.transpose(nx2, (1, 0, 2)).reshape(3, f * s2)
    out = _sa3(x3p, l2, params['sa3'], nf=f)
    return out.reshape(b, t, -1)


# speedup vs baseline: 1.8559x; 1.1393x over previous
import functools

import jax
import jax.numpy as jnp
from jax import lax
from jax.experimental import pallas as pl
from jax.experimental.pallas import tpu as pltpu
from jax.experimental.pallas import tpu_sc as plsc

_NC = 2
_NS = 16
_NW = _NC * _NS

_BN_EPS = 1e-5


def _fps_body(npoint, x_ref, o_ref):
    x = x_ref[...]
    xx, yy, zz = x[:, 0], x[:, 1], x[:, 2]
    f, ns, nl = xx.shape
    flat = (lax.broadcasted_iota(jnp.int32, (f, ns, nl), 1) * nl
            + lax.broadcasted_iota(jnp.int32, (f, ns, nl), 2))
    big = jnp.int32(2 ** 30)

    col = lax.broadcasted_iota(jnp.int32, (1, 1, npoint), 2)
    row = lax.broadcasted_iota(jnp.int32, (1, 3, 1), 1)

    def step(i, carry):
        far, dist, acc = carry
        selm = flat == far
        cx = jnp.sum(jnp.where(selm, xx, 0.0), axis=(1, 2), keepdims=True)
        cy = jnp.sum(jnp.where(selm, yy, 0.0), axis=(1, 2), keepdims=True)
        cz = jnp.sum(jnp.where(selm, zz, 0.0), axis=(1, 2), keepdims=True)
        d = ((xx - cx) ** 2 + (yy - cy) ** 2) + (zz - cz) ** 2
        dist = jnp.minimum(dist, d)
        m = jnp.max(dist, axis=(1, 2), keepdims=True)
        far2 = jnp.min(jnp.where(dist == m, flat, big), axis=(1, 2), keepdims=True)
        cb = jnp.where(row == 0, cx, jnp.where(row == 1, cy, cz))
        acc = jnp.where(col == i, cb, acc)
        return far2, dist, acc

    init = (jnp.zeros((f, 1, 1), jnp.int32),
            jnp.full((f, ns, nl), 1e10, jnp.float32),
            jnp.zeros((f, 3, npoint), jnp.float32))
    _, _, acc = lax.fori_loop(0, npoint, step, init)
    o_ref[...] = acc


def _fps(xc, npoint):
    f, _, ns, nl = xc.shape
    return pl.pallas_call(
        functools.partial(_fps_body, npoint),
        out_shape=jax.ShapeDtypeStruct((f, 3, npoint), jnp.float32),
    )(xc)


def _stats_update(acc, s_ref, h, pid):
    @pl.when(pid == 0)
    def _():
        acc[...] = jnp.zeros_like(acc)

    acc[0:1, :] += jnp.sum(h, axis=0, keepdims=True)
    acc[1:2, :] += jnp.sum(h * h, axis=0, keepdims=True)
    s_ref[...] = acc[...]


def _affine(s, g, bt, rcount):
    mean = s[0:1, :] / rcount
    var = s[1:2, :] / rcount - mean * mean
    a = g * lax.rsqrt(var + _BN_EPS)
    return a, bt - mean * a


def _mmp_body(x_ref, w_ref, b_ref, h_ref, s_ref, acc):
    h = lax.dot_general(x_ref[...], w_ref[...], (((0,), (0,)), ((), ())),
                        preferred_element_type=jnp.float32) + b_ref[...]
    h_ref[...] = h
    _stats_update(acc, s_ref, h, pl.program_id(0))


def _mmp(xp, w, b, rb):
    c, r = xp.shape
    d = w.shape[1]
    return pl.pallas_call(
        _mmp_body,
        grid=(r // rb,),
        in_specs=[
            pl.BlockSpec((c, rb), lambda i: (0, i)),
            pl.BlockSpec((c, d), lambda i: (0, 0)),
            pl.BlockSpec((1, d), lambda i: (0, 0)),
        ],
        out_specs=[
            pl.BlockSpec((rb, d), lambda i: (i, 0)),
            pl.BlockSpec((2, d), lambda i: (0, 0)),
        ],
        out_shape=[jax.ShapeDtypeStruct((r, d), jnp.float32),
                   jax.ShapeDtypeStruct((2, d), jnp.float32)],
        scratch_shapes=[pltpu.VMEM((2, d), jnp.float32)],
    )(xp, w, b.reshape(1, d))


def _mmd_body(xp_ref, xr_ref, wa_ref, wb_ref, b_ref, h_ref, s_ref, acc):
    h = (lax.dot_general(xp_ref[...], wa_ref[...], (((0,), (0,)), ((), ())),
                         preferred_element_type=jnp.float32)
         + jnp.dot(xr_ref[...], wb_ref[...], preferred_element_type=jnp.float32)
         + b_ref[...])
    h_ref[...] = h
    _stats_update(acc, s_ref, h, pl.program_id(0))


def _mmd(xp, xr, w, b, rb):
    cp, r = xp.shape
    cr = xr.shape[1]
    d = w.shape[1]
    return pl.pallas_call(
        _mmd_body,
        grid=(r // rb,),
        in_specs=[
            pl.BlockSpec((cp, rb), lambda i: (0, i)),
            pl.BlockSpec((rb, cr), lambda i: (i, 0)),
            pl.BlockSpec((cp, d), lambda i: (0, 0)),
            pl.BlockSpec((cr, d), lambda i: (0, 0)),
            pl.BlockSpec((1, d), lambda i: (0, 0)),
        ],
        out_specs=[
            pl.BlockSpec((rb, d), lambda i: (i, 0)),
            pl.BlockSpec((2, d), lambda i: (0, 0)),
        ],
        out_shape=[jax.ShapeDtypeStruct((r, d), jnp.float32),
                   jax.ShapeDtypeStruct((2, d), jnp.float32)],
        scratch_shapes=[pltpu.VMEM((2, d), jnp.float32)],
    )(xp, xr, w[:cp], w[cp:], b.reshape(1, d))


def _mmr_body(rcount, x_ref, sin_ref, g_ref, bt_ref, w_ref, b_ref,
              h_ref, s_ref, acc):
    a, c = _affine(sin_ref[...], g_ref[...], bt_ref[...], rcount)
    x = jnp.maximum(x_ref[...] * a + c, 0.0)
    h = jnp.dot(x, w_ref[...], preferred_element_type=jnp.float32) + b_ref[...]
    h_ref[...] = h
    _stats_update(acc, s_ref, h, pl.program_id(0))


def _mmr(x, sin, g, bt, w, b, rb):
    r, cin = x.shape
    d = w.shape[1]
    return pl.pallas_call(
        functools.partial(_mmr_body, float(r)),
        grid=(r // rb,),
        in_specs=[
            pl.BlockSpec((rb, cin), lambda i: (i, 0)),
            pl.BlockSpec((2, cin), lambda i: (0, 0)),
            pl.BlockSpec((1, cin), lambda i: (0, 0)),
            pl.BlockSpec((1, cin), lambda i: (0, 0)),
            pl.BlockSpec((cin, d), lambda i: (0, 0)),
            pl.BlockSpec((1, d), lambda i: (0, 0)),
        ],
        out_specs=[
            pl.BlockSpec((rb, d), lambda i: (i, 0)),
            pl.BlockSpec((2, d), lambda i: (0, 0)),
        ],
        out_shape=[jax.ShapeDtypeStruct((r, d), jnp.float32),
                   jax.ShapeDtypeStruct((2, d), jnp.float32)],
        scratch_shapes=[pltpu.VMEM((2, d), jnp.float32)],
    )(x, sin, g.reshape(1, cin), bt.reshape(1, cin), w, b.reshape(1, d))


def _pool_body(rcount, k, x_ref, sin_ref, g_ref, bt_ref, o_ref):
    a, c = _affine(sin_ref[...], g_ref[...], bt_ref[...], rcount)
    h = jnp.maximum(x_ref[...] * a + c, 0.0)
    rb, d = h.shape
    o_ref[...] = jnp.max(h.reshape(rb // k, k, d), axis=1)


def _pool(x, sin, g, bt, k, rb):
    r, d = x.shape
    return pl.pallas_call(
        functools.partial(_pool_body, float(r), k),
        grid=(r // rb,),
        in_specs=[
            pl.BlockSpec((rb, d), lambda i: (i, 0)),
            pl.BlockSpec((2, d), lambda i: (0, 0)),
            pl.BlockSpec((1, d), lambda i: (0, 0)),
            pl.BlockSpec((1, d), lambda i: (0, 0)),
        ],
        out_specs=pl.BlockSpec((rb // k, d), lambda i: (i, 0)),
        out_shape=jax.ShapeDtypeStruct((r // k, d), jnp.float32),
    )(x, sin, g.reshape(1, d), bt.reshape(1, d))


def _bn_relu_full(h, g, bt):
    mean = jnp.mean(h, axis=0, keepdims=True)
    var = jnp.mean(h * h, axis=0, keepdims=True) - mean * mean
    a = g * lax.rsqrt(var + _BN_EPS)
    return jnp.maximum(h * a + (bt - mean * a), 0.0)


def _sa3_body(nf, xp_ref, xr_ref, w1a_ref, w1b_ref, b1_ref, g1_ref, t1_ref,
              w2_ref, b2_ref, g2_ref, t2_ref, w3_ref, b3_ref, g3_ref, t3_ref,
              o_ref):
    h = (lax.dot_general(xp_ref[...], w1a_ref[...], (((0,), (0,)), ((), ())),
                         preferred_element_type=jnp.float32)
         + jnp.dot(xr_ref[...], w1b_ref[...], preferred_element_type=jnp.float32)
         + b1_ref[...])
    h = _bn_relu_full(h, g1_ref[...], t1_ref[...])
    h = jnp.dot(h, w2_ref[...], preferred_element_type=jnp.float32) + b2_ref[...]
    h = _bn_relu_full(h, g2_ref[...], t2_ref[...])
    h = jnp.dot(h, w3_ref[...], preferred_element_type=jnp.float32) + b3_ref[...]
    h = _bn_relu_full(h, g3_ref[...], t3_ref[...])
    r, d = h.shape
    o_ref[...] = jnp.max(h.reshape(nf, r // nf, d), axis=1)


def _sa3(xp, xr, params, nf):
    (w1, b1, g1, t1), (w2, b2, g2, t2), (w3, b3, g3, t3) = params
    r = xr.shape[0]
    d = w3.shape[1]
    args = [xp, xr, w1[:3], w1[3:], b1.reshape(1, -1), g1.reshape(1, -1),
            t1.reshape(1, -1), w2, b2.reshape(1, -1), g2.reshape(1, -1),
            t2.reshape(1, -1), w3, b3.reshape(1, -1), g3.reshape(1, -1),
            t3.reshape(1, -1)]
    return pl.pallas_call(
        functools.partial(_sa3_body, nf),
        out_shape=jax.ShapeDtypeStruct((nf, d), jnp.float32),
    )(*args)


def _mesh():
    return plsc.VectorSubcoreMesh(core_axis_name="c", subcore_axis_name="s")


def _rnd16(v):
    b = plsc.bitcast(v, jnp.uint32)
    b = (b + jnp.uint32(0x7FFF) + ((b >> jnp.uint32(16)) & jnp.uint32(1))) & jnp.uint32(0xFFFF0000)
    return plsc.bitcast(b, jnp.float32)


def _ball1(ptsT, cent):
    f, _, n = ptsT.shape
    s = cent.shape[2]
    k = 32
    r2 = 0.2 ** 2
    wpf = _NW // f
    cpw = s // wpf
    nblk = n // 16

    @functools.partial(
        pl.kernel, mesh=_mesh(),
        compiler_params=pltpu.CompilerParams(needs_layout_passes=False),
        out_type=jax.ShapeDtypeStruct((f * 4 * s * k,), jnp.float32),
        scratch_types=[pltpu.VMEM((n,), jnp.float32)] * 8
                      + [pltpu.VMEM((max(cpw, 128),), jnp.float32)] * 3
                      + [pltpu.VMEM((cpw * k,), jnp.float32)] * 4
                      + [pltpu.VMEM((128,), jnp.int32)],
    )
    def run(pts_h, cent_h, g_h, px, py, pz, pt, bx, by, bz, nd_s,
            cx, cy, cz, ob0, ob1, ob2, ob3, ib):
        wid = lax.axis_index("s") * _NC + lax.axis_index("c")
        fr = wid // wpf
        c0 = (wid % wpf) * cpw
        pltpu.sync_copy(pts_h.at[pl.ds((fr * 4 + 0) * n, n)], px)
        pltpu.sync_copy(pts_h.at[pl.ds((fr * 4 + 1) * n, n)], py)
        pltpu.sync_copy(pts_h.at[pl.ds((fr * 4 + 2) * n, n)], pz)
        pltpu.sync_copy(pts_h.at[pl.ds((fr * 4 + 3) * n, n)], pt)
        pltpu.sync_copy(cent_h.at[pl.ds((fr * 3 + 0) * s + c0, cpw)], cx.at[pl.ds(0, cpw)])
        pltpu.sync_copy(cent_h.at[pl.ds((fr * 3 + 1) * s + c0, cpw)], cy.at[pl.ds(0, cpw)])
        pltpu.sync_copy(cent_h.at[pl.ds((fr * 3 + 2) * s + c0, cpw)], cz.at[pl.ds(0, cpw)])
        iota = lax.iota(jnp.int32, 16)
        zeros16 = jnp.zeros((16,), jnp.int32)

        def pre(jj, _):
            sl = pl.ds(jj * 16, 16)
            xv, yv, zv = px[sl], py[sl], pz[sl]
            bx[sl] = _rnd16(xv)
            by[sl] = _rnd16(yv)
            bz[sl] = _rnd16(zv)
            nd_s[sl] = (xv * xv + yv * yv) + zv * zv
            return 0

        lax.fori_loop(0, nblk, pre, 0)

        def per_cent(ci, _):
            cis = zeros16 + ci
            cxs = plsc.load_gather(cx, [cis])
            cys = plsc.load_gather(cy, [cis])
            czs = plsc.load_gather(cz, [cis])
            ns = (cxs * cxs + cys * cys) + czs * czs
            cxb = _rnd16(cxs)
            cyb = _rnd16(cys)
            czb = _rnd16(czs)

            def cond(st):
                j, cntv = st
                return jnp.logical_and(jnp.any(cntv < k), j < nblk // 4)

            def bodyw(st):
                j, cntv = st
                off = cntv
                for b in range(4):
                    base = j * 64 + b * 16
                    sl = pl.ds(base, 16)
                    pidx = base + iota
                    dot = (bx[sl] * cxb + by[sl] * cyb) + bz[sl] * czb
                    d = (ns + nd_s[sl]) - 2.0 * dot
                    mask = d <= r2
                    mi = mask.astype(jnp.int32)
                    inc = plsc.cumsum(mi)
                    plsc.store_scatter(ib, [off + (inc - mi)], pidx, mask=mask)
                    off = off + plsc.all_reduce_population_count(mask)
                return j + 1, off

            _, cnt = lax.while_loop(
                cond, bodyw, (jnp.int32(0), jnp.zeros((16,), jnp.int32)))
            first = plsc.load_gather(ib, [jnp.where(cnt < 0, iota, zeros16)])
            for h in range(k // 16):
                pos = h * 16 + iota
                iv = plsc.load_gather(ib, [pos])
                sel = jnp.where(pos >= cnt, first, iv)
                opos = ci * k + pos
                plsc.store_scatter(ob0, [opos], plsc.load_gather(px, [sel]) - cxs)
                plsc.store_scatter(ob1, [opos], plsc.load_gather(py, [sel]) - cys)
                plsc.store_scatter(ob2, [opos], plsc.load_gather(pz, [sel]) - czs)
                plsc.store_scatter(ob3, [opos], plsc.load_gather(pt, [sel]))
            return 0

        lax.fori_loop(0, cpw, per_cent, 0)
        sk = s * k
        pltpu.sync_copy(ob0, g_h.at[pl.ds((fr * 4 + 0) * sk + c0 * k, cpw * k)])
        pltpu.sync_copy(ob1, g_h.at[pl.ds((fr * 4 + 1) * sk + c0 * k, cpw * k)])
        pltpu.sync_copy(ob2, g_h.at[pl.ds((fr * 4 + 2) * sk + c0 * k, cpw * k)])
        pltpu.sync_copy(ob3, g_h.at[pl.ds((fr * 4 + 3) * sk + c0 * k, cpw * k)])

    return run(ptsT.reshape(-1), cent.reshape(-1)).reshape(f, 4, s * k)


def _ball2(xyzT, cent):
    f, _, n = xyzT.shape
    s = cent.shape[2]
    k = 64
    r2 = 0.4 ** 2
    wpf = _NW // f
    cpw = s // wpf
    nblk = n // 16

    @functools.partial(
        pl.kernel, mesh=_mesh(),
        compiler_params=pltpu.CompilerParams(needs_layout_passes=False),
        out_type=[jax.ShapeDtypeStruct((f * 3 * s * k,), jnp.float32),
                  jax.ShapeDtypeStruct((f * s * k,), jnp.int32)],
        scratch_types=[pltpu.VMEM((n,), jnp.float32)] * 7
                      + [pltpu.VMEM((max(cpw, 128),), jnp.float32)] * 3
                      + [pltpu.VMEM((cpw * k,), jnp.float32)] * 3
                      + [pltpu.VMEM((cpw * k,), jnp.int32),
                         pltpu.VMEM((128,), jnp.int32)],
    )
    def run(pts_h, cent_h, g_h, ix_h, px, py, pz, bx, by, bz, nd_s,
            cx, cy, cz, ob0, ob1, ob2, obi, ib):
        wid = lax.axis_index("s") * _NC + lax.axis_index("c")
        fr = wid // wpf
        c0 = (wid % wpf) * cpw
        pltpu.sync_copy(pts_h.at[pl.ds((fr * 3 + 0) * n, n)], px)
        pltpu.sync_copy(pts_h.at[pl.ds((fr * 3 + 1) * n, n)], py)
        pltpu.sync_copy(pts_h.at[pl.ds((fr * 3 + 2) * n, n)], pz)
        pltpu.sync_copy(cent_h.at[pl.ds((fr * 3 + 0) * s + c0, cpw)], cx.at[pl.ds(0, cpw)])
        pltpu.sync_copy(cent_h.at[pl.ds((fr * 3 + 1) * s + c0, cpw)], cy.at[pl.ds(0, cpw)])
        pltpu.sync_copy(cent_h.at[pl.ds((fr * 3 + 2) * s + c0, cpw)], cz.at[pl.ds(0, cpw)])
        iota = lax.iota(jnp.int32, 16)
        zeros16 = jnp.zeros((16,), jnp.int32)

        def pre(jj, _):
            sl = pl.ds(jj * 16, 16)
            xv, yv, zv = px[sl], py[sl], pz[sl]
            bx[sl] = _rnd16(xv)
            by[sl] = _rnd16(yv)
            bz[sl] = _rnd16(zv)
            nd_s[sl] = (xv * xv + yv * yv) + zv * zv
            return 0

        lax.fori_loop(0, nblk, pre, 0)

        def per_cent(ci, _):
            cis = zeros16 + ci
            cxs = plsc.load_gather(cx, [cis])
            cys = plsc.load_gather(cy, [cis])
            czs = plsc.load_gather(cz, [cis])
            ns = (cxs * cxs + cys * cys) + czs * czs
            cxb = _rnd16(cxs)
            cyb = _rnd16(cys)
            czb = _rnd16(czs)

            def cond(st):
                j, cntv = st
                return jnp.logical_and(jnp.any(cntv < k), j < nblk // 4)

            def bodyw(st):
                j, cntv = st
                off = cntv
                for b in range(4):
                    base = j * 64 + b * 16
                    sl = pl.ds(base, 16)
                    pidx = base + iota
                    dot = (bx[sl] * cxb + by[sl] * cyb) + bz[sl] * czb
                    d = (ns + nd_s[sl]) - 2.0 * dot
                    mask = d <= r2
                    mi = mask.astype(jnp.int32)
                    inc = plsc.cumsum(mi)
                    plsc.store_scatter(ib, [off + (inc - mi)], pidx, mask=mask)
                    off = off + plsc.all_reduce_population_count(mask)
                return j + 1, off

            _, cnt = lax.while_loop(
                cond, bodyw, (jnp.int32(0), jnp.zeros((16,), jnp.int32)))
            first = plsc.load_gather(ib, [jnp.where(cnt < 0, iota, zeros16)])
            for h in range(k // 16):
                pos = h * 16 + iota
                iv = plsc.load_gather(ib, [pos])
                sel = jnp.where(pos >= cnt, first, iv)
                opos = ci * k + pos
                plsc.store_scatter(ob0, [opos], plsc.load_gather(px, [sel]) - cxs)
                plsc.store_scatter(ob1, [opos], plsc.load_gather(py, [sel]) - cys)
                plsc.store_scatter(ob2, [opos], plsc.load_gather(pz, [sel]) - czs)
                plsc.store_scatter(obi, [opos], sel + fr * n)
            return 0

        lax.fori_loop(0, cpw, per_cent, 0)
        sk = s * k
        pltpu.sync_copy(ob0, g_h.at[pl.ds((fr * 3 + 0) * sk + c0 * k, cpw * k)])
        pltpu.sync_copy(ob1, g_h.at[pl.ds((fr * 3 + 1) * sk + c0 * k, cpw * k)])
        pltpu.sync_copy(ob2, g_h.at[pl.ds((fr * 3 + 2) * sk + c0 * k, cpw * k)])
        pltpu.sync_copy(obi, ix_h.at[pl.ds(wid * cpw * k, cpw * k)])

    g_out, ix_out = run(xyzT.reshape(-1), cent.reshape(-1))
    return g_out.reshape(f, 3, s * k), ix_out


def _gatherk(table, idx):
    v, d = table.shape
    b = idx.shape[0]
    bpw = b // _NW
    chunk = 128
    nch = bpw // chunk

    @functools.partial(
        pl.kernel, mesh=_mesh(),
        compiler_params=pltpu.CompilerParams(needs_layout_passes=False),
        out_type=jax.ShapeDtypeStruct((b, d), jnp.float32),
        scratch_types=[pltpu.VMEM((chunk,), jnp.int32),
                       pltpu.VMEM((chunk, d), jnp.float32),
                       pltpu.SemaphoreType.DMA],
    )
    def run(tbl_h, idx_h, out_h, idx_v, rows_v, sem):
        wid = lax.axis_index("s") * _NC + lax.axis_index("c")
        base = wid * bpw

        def body(i, _):
            off = base + i * chunk
            pltpu.sync_copy(idx_h.at[pl.ds(off, chunk)], idx_v)
            pltpu.async_copy(tbl_h.at[idx_v], rows_v, sem).wait()
            pltpu.sync_copy(rows_v, out_h.at[pl.ds(off, chunk)])
            return 0

        lax.fori_loop(0, nch, body, 0)

    return run(table, idx)


def kernel(xyz_time, params):
    b, t, n, _ = xyz_time.shape
    f = b * t
    xt = xyz_time.reshape(f, n, 4)
    ptsT = jnp.transpose(xt, (0, 2, 1))
    xyzT = ptsT[:, :3]

    s1, k1 = 512, 32
    nx1 = _fps(xyzT.reshape(f, 3, 8, n // 8), s1)
    g1 = _ball1(ptsT, nx1)
    x1p = jnp.transpose(g1, (1, 0, 2)).reshape(4, f * s1 * k1)
    (w1, b1, g1p, t1p), (w2, b2, g2p, t2p), (w3, b3, g3p, t3p) = params['sa1']
    h1, st1 = _mmp(x1p, w1, b1, rb=8192)
    h2, st2 = _mmr(h1, st1, g1p, t1p, w2, b2, rb=8192)
    h3, st3 = _mmr(h2, st2, g2p, t2p, w3, b3, rb=8192)
    l1 = _pool(h3, st3, g3p, t3p, k=k1, rb=8192)

    s2, k2 = 128, 64
    nx2 = _fps(nx1.reshape(f, 3, 8, s1 // 8), s2)
    g2, idxg = _ball2(nx1, nx2)
    feats = _gatherk(l1, idxg)
    x2p = jnp.transpose(g2, (1, 0, 2)).reshape(3, f * s2 * k2)
    (w1, b1, g1p, t1p), (w2, b2, g2p, t2p), (w3, b3, g3p, t3p) = params['sa2']
    h1, st1 = _mmd(x2p, feats, w1, b1, rb=8192)
    h2, st2 = _mmr(h1, st1, g1p, t1p, w2, b2, rb=8192)
    h3, st3 = _mmr(h2, st2, g2p, t2p, w3, b3, rb=8192)
    l2 = _pool(h3, st3, g3p, t3p, k=k2, rb=8192)

    x3p = jnp.transpose(nx2, (1, 0, 2)).reshape(3, f * s2)
    out = _sa3(x3p, l2, params['sa3'], nf=f)
    return out.reshape(b, t, -1)
